# baseline (speedup 1.0000x reference)
import functools

import jax
import jax.numpy as jnp
from jax import lax
from jax.experimental import pallas as pl
from jax.experimental.pallas import tpu as pltpu
from jax.experimental.pallas import tpu_sc as plsc

_N_CHANNELS = 4096
_N_BINS = 8192
_LANES = 16

_TC_ROWS = 4096
_SC_ROWS = _N_CHANNELS - _TC_ROWS

_TC_NBUF = 6

_ROW_BLOCK = 256
_NW = 32
_SC_ROWS_PER_W = _SC_ROWS // _NW if _SC_ROWS else 0
_RCHUNK = 4
_NBUF = 2

_LN2 = 0.6931471805599453


def _tc_mv_body(params_ref, energies_ref, m_ref, out_ref, flux_ref):
    i = pl.program_id(0)

    @pl.when(i == 0)
    def _():
        alpha = params_ref[0]
        norm = params_ref[1]
        oma = 1.0 - alpha
        e_low = energies_ref[0, :]
        e_high = energies_ref[1, :]
        flux_ref[0, :] = norm * (jnp.exp(oma * jnp.log(e_high))
                                 - jnp.exp(oma * jnp.log(e_low))) / oma

    m = m_ref[...]
    acc = jnp.sum(m * flux_ref[0, :][None, :], axis=1)
    out_ref[0, 0, :] = jnp.maximum(acc, 1e-6)


def _tc_mv_ring_body(params_ref, energies_ref, tm_ref, out_ref,
                     ring, flux_ref, sem):
    n_chunks = _TC_ROWS // _ROW_BLOCK
    alpha = params_ref[0]
    norm = params_ref[1]
    oma = 1.0 - alpha
    e_low = energies_ref[0, :]
    e_high = energies_ref[1, :]
    flux = norm * (jnp.exp(oma * jnp.log(e_high))
                   - jnp.exp(oma * jnp.log(e_low))) / oma
    flux_ref[0, :] = flux

    def chunk_copy(k, b):
        src = tm_ref.at[pl.ds(k * _ROW_BLOCK, _ROW_BLOCK), :]
        return pltpu.make_async_copy(src, ring.at[b], sem.at[b])

    for b in range(_TC_NBUF):
        chunk_copy(b, b).start()
    for k in range(n_chunks):
        b = k % _TC_NBUF
        chunk_copy(k, b).wait()
        acc = jnp.sum(ring[b] * flux_ref[0, :][None, :], axis=1)
        out_ref[pl.ds(k * _ROW_BLOCK, _ROW_BLOCK)] = jnp.maximum(acc, 1e-6)
        if k + _TC_NBUF < n_chunks:
            chunk_copy(k + _TC_NBUF, b).start()


def _tc_matvec(parameters, energies, transfer_matrix):
    return pl.pallas_call(
        _tc_mv_ring_body,
        in_specs=[
            pl.BlockSpec(memory_space=pltpu.SMEM),
            pl.BlockSpec((2, _N_BINS), lambda: (0, 0)),
            pl.BlockSpec(memory_space=pl.MemorySpace.ANY),
        ],
        out_specs=pl.BlockSpec(memory_space=pltpu.VMEM),
        out_shape=jax.ShapeDtypeStruct((_TC_ROWS,), jnp.float32),
        scratch_shapes=[
            pltpu.VMEM((_TC_NBUF, _ROW_BLOCK, _N_BINS), jnp.float32),
            pltpu.VMEM((1, _N_BINS), jnp.float32),
            pltpu.SemaphoreType.DMA((_TC_NBUF,)),
        ],
    )(parameters, energies, transfer_matrix)


def _sc_log2(x):
    bits = plsc.bitcast(x, jnp.int32)
    expo = lax.shift_right_logical(bits, 23) - 127
    m = plsc.bitcast((bits & 0x007FFFFF) | 0x3F800000, jnp.float32)
    t = (m - 1.0) / (m + 1.0)
    t2 = t * t
    p = t2 * (1.0 / 11.0)
    p = t2 * (p + 1.0 / 9.0)
    p = t2 * (p + 1.0 / 7.0)
    p = t2 * (p + 1.0 / 5.0)
    p = t2 * (p + 1.0 / 3.0)
    log2m = t * (p + 1.0) * (2.0 / _LN2)
    return expo.astype(jnp.float32) + log2m


def _sc_mv_body(params_hbm, en_hbm, tm_hbm, out_hbm,
                params_v, en_v, flux_v, ring, out_v, sem0, sem1):
    n_chunks = _SC_ROWS_PER_W // _RCHUNK
    group = _LANES // _RCHUNK
    wid = lax.axis_index("s") * 2 + lax.axis_index("c")
    row0 = _TC_ROWS + wid * _SC_ROWS_PER_W
    sems = (sem0, sem1)
    iota16 = lax.iota(jnp.int32, _LANES)

    def chunk_copy(k, b):
        src = tm_hbm.at[pl.ds(row0 + k * _RCHUNK, _RCHUNK), :]
        return pltpu.make_async_copy(src, ring.at[b], sems[b])

    for b in range(_NBUF):
        chunk_copy(b, b).start()

    pltpu.sync_copy(params_hbm, params_v)
    pltpu.sync_copy(en_hbm, en_v)
    pv = params_v[pl.ds(0, _LANES)]
    alpha = pv[0]
    norm = pv[1]
    oma = 1.0 - alpha
    c_exp = oma * _LN2
    scale = (jnp.zeros((_LANES,), jnp.float32) + norm) / \
        (jnp.zeros((_LANES,), jnp.float32) + oma)

    def fbody(j, carry):
        col = pl.multiple_of(j * _LANES, _LANES)
        e_low = en_v[0, pl.ds(col, _LANES)]
        e_high = en_v[1, pl.ds(col, _LANES)]
        p_hi = jnp.exp(c_exp * _sc_log2(e_high))
        p_lo = jnp.exp(c_exp * _sc_log2(e_low))
        flux_v[pl.ds(col, _LANES)] = scale * (p_hi - p_lo)
        return carry

    lax.fori_loop(0, _N_BINS // _LANES, fbody, 0)

    def do_chunk(k, b, g, res):
        chunk_copy(k, b).wait()
        zero = jnp.zeros((_LANES,), jnp.float32)

        def inner(j, accs):
            col = pl.multiple_of(j * _LANES, _LANES)
            f = flux_v[pl.ds(col, _LANES)]
            return tuple(acc + ring[b, i, pl.ds(col, _LANES)] * f
                         for i, acc in enumerate(accs))

        accs = lax.fori_loop(0, _N_BINS // _LANES, inner,
                             (zero,) * _RCHUNK)
        for i in range(_RCHUNK):
            s = plsc.cumsum(accs[i])[_LANES - 1]
            res = jnp.where(iota16 == g * _RCHUNK + i, s, res)

        @pl.when(k + _NBUF < n_chunks)
        def _():
            chunk_copy(k + _NBUF, b).start()
        return res

    def outer(grp, carry):
        k0 = grp * group
        res = jnp.zeros((_LANES,), jnp.float32)
        for j in range(group):
            res = do_chunk(k0 + j, j % _NBUF, j, res)
        base = pl.multiple_of(k0 * _RCHUNK, _LANES)
        out_v[pl.ds(base, _LANES)] = jnp.maximum(res, 1e-6)
        return carry

    lax.fori_loop(0, n_chunks // group, outer, 0)
    pltpu.sync_copy(out_v, out_hbm.at[pl.ds(wid * _SC_ROWS_PER_W,
                                            _SC_ROWS_PER_W)])


def _sc_matvec(parameters, energies, transfer_matrix):
    params16 = jnp.zeros((_LANES,), jnp.float32).at[:2].set(parameters)
    mesh = plsc.VectorSubcoreMesh(core_axis_name="c", subcore_axis_name="s")
    kern = functools.partial(
        pl.kernel,
        out_type=jax.ShapeDtypeStruct((_SC_ROWS,), jnp.float32),
        mesh=mesh,
        compiler_params=pltpu.CompilerParams(needs_layout_passes=False),
        scratch_types=[
            pltpu.VMEM((_LANES,), jnp.float32),
            pltpu.VMEM((2, _N_BINS), jnp.float32),
            pltpu.VMEM((_N_BINS,), jnp.float32),
            pltpu.VMEM((_NBUF, _RCHUNK, _N_BINS), jnp.float32),
            pltpu.VMEM((_SC_ROWS_PER_W,), jnp.float32),
            pltpu.SemaphoreType.DMA,
            pltpu.SemaphoreType.DMA,
        ],
    )(_sc_mv_body)
    return kern(params16, energies, transfer_matrix)


def kernel(parameters, energies, transfer_matrix):
    parts = []
    if _SC_ROWS:
        sc_part = _sc_matvec(parameters, energies, transfer_matrix)
    if _TC_ROWS:
        parts.append(_tc_matvec(parameters, energies, transfer_matrix))
    if _SC_ROWS:
        parts.append(sc_part)
    if len(parts) == 1:
        return parts[0]
    return jnp.concatenate(parts)


# speedup vs baseline: 1.0573x; 1.0573x over previous
import functools

import jax
import jax.numpy as jnp
from jax import lax
from jax.experimental import pallas as pl
from jax.experimental.pallas import tpu as pltpu
from jax.experimental.pallas import tpu_sc as plsc

_N_CHANNELS = 4096
_N_BINS = 8192
_LANES = 16

_TC_ROWS = 4096
_SC_ROWS = _N_CHANNELS - _TC_ROWS

_TC_NBUF = 16

_ROW_BLOCK = 64
_NW = 32
_SC_ROWS_PER_W = _SC_ROWS // _NW if _SC_ROWS else 0
_RCHUNK = 4
_NBUF = 2

_LN2 = 0.6931471805599453


def _tc_mv_body(params_ref, energies_ref, m_ref, out_ref, flux_ref):
    i = pl.program_id(0)

    @pl.when(i == 0)
    def _():
        alpha = params_ref[0]
        norm = params_ref[1]
        oma = 1.0 - alpha
        e_low = energies_ref[0, :]
        e_high = energies_ref[1, :]
        flux_ref[0, :] = norm * (jnp.exp(oma * jnp.log(e_high))
                                 - jnp.exp(oma * jnp.log(e_low))) / oma

    m = m_ref[...]
    acc = jnp.sum(m * flux_ref[0, :][None, :], axis=1)
    out_ref[0, 0, :] = jnp.maximum(acc, 1e-6)


def _tc_mv_ring_body(params_ref, energies_ref, tm_ref, out_ref,
                     ring, flux_ref, sem):
    n_chunks = _TC_ROWS // _ROW_BLOCK
    alpha = params_ref[0]
    norm = params_ref[1]
    oma = 1.0 - alpha
    e_low = energies_ref[0, :]
    e_high = energies_ref[1, :]
    flux = norm * (jnp.exp(oma * jnp.log(e_high))
                   - jnp.exp(oma * jnp.log(e_low))) / oma
    flux_ref[0, :] = flux

    def chunk_copy(k, b):
        src = tm_ref.at[pl.ds(k * _ROW_BLOCK, _ROW_BLOCK), :]
        return pltpu.make_async_copy(src, ring.at[b], sem.at[b])

    for b in range(_TC_NBUF):
        chunk_copy(b, b).start()
    for k in range(n_chunks):
        b = k % _TC_NBUF
        chunk_copy(k, b).wait()
        acc = jnp.sum(ring[b] * flux_ref[0, :][None, :], axis=1)
        out_ref[pl.ds(k * _ROW_BLOCK, _ROW_BLOCK)] = jnp.maximum(acc, 1e-6)
        if k + _TC_NBUF < n_chunks:
            chunk_copy(k + _TC_NBUF, b).start()


def _tc_matvec(parameters, energies, transfer_matrix):
    return pl.pallas_call(
        _tc_mv_ring_body,
        in_specs=[
            pl.BlockSpec(memory_space=pltpu.SMEM),
            pl.BlockSpec((2, _N_BINS), lambda: (0, 0)),
            pl.BlockSpec(memory_space=pl.MemorySpace.ANY),
        ],
        out_specs=pl.BlockSpec(memory_space=pltpu.VMEM),
        out_shape=jax.ShapeDtypeStruct((_TC_ROWS,), jnp.float32),
        scratch_shapes=[
            pltpu.VMEM((_TC_NBUF, _ROW_BLOCK, _N_BINS), jnp.float32),
            pltpu.VMEM((1, _N_BINS), jnp.float32),
            pltpu.SemaphoreType.DMA((_TC_NBUF,)),
        ],
    )(parameters, energies, transfer_matrix)


def _sc_log2(x):
    bits = plsc.bitcast(x, jnp.int32)
    expo = lax.shift_right_logical(bits, 23) - 127
    m = plsc.bitcast((bits & 0x007FFFFF) | 0x3F800000, jnp.float32)
    t = (m - 1.0) / (m + 1.0)
    t2 = t * t
    p = t2 * (1.0 / 11.0)
    p = t2 * (p + 1.0 / 9.0)
    p = t2 * (p + 1.0 / 7.0)
    p = t2 * (p + 1.0 / 5.0)
    p = t2 * (p + 1.0 / 3.0)
    log2m = t * (p + 1.0) * (2.0 / _LN2)
    return expo.astype(jnp.float32) + log2m


def _sc_mv_body(params_hbm, en_hbm, tm_hbm, out_hbm,
                params_v, en_v, flux_v, ring, out_v, sem0, sem1):
    n_chunks = _SC_ROWS_PER_W // _RCHUNK
    group = _LANES // _RCHUNK
    wid = lax.axis_index("s") * 2 + lax.axis_index("c")
    row0 = _TC_ROWS + wid * _SC_ROWS_PER_W
    sems = (sem0, sem1)
    iota16 = lax.iota(jnp.int32, _LANES)

    def chunk_copy(k, b):
        src = tm_hbm.at[pl.ds(row0 + k * _RCHUNK, _RCHUNK), :]
        return pltpu.make_async_copy(src, ring.at[b], sems[b])

    for b in range(_NBUF):
        chunk_copy(b, b).start()

    pltpu.sync_copy(params_hbm, params_v)
    pltpu.sync_copy(en_hbm, en_v)
    pv = params_v[pl.ds(0, _LANES)]
    alpha = pv[0]
    norm = pv[1]
    oma = 1.0 - alpha
    c_exp = oma * _LN2
    scale = (jnp.zeros((_LANES,), jnp.float32) + norm) / \
        (jnp.zeros((_LANES,), jnp.float32) + oma)

    def fbody(j, carry):
        col = pl.multiple_of(j * _LANES, _LANES)
        e_low = en_v[0, pl.ds(col, _LANES)]
        e_high = en_v[1, pl.ds(col, _LANES)]
        p_hi = jnp.exp(c_exp * _sc_log2(e_high))
        p_lo = jnp.exp(c_exp * _sc_log2(e_low))
        flux_v[pl.ds(col, _LANES)] = scale * (p_hi - p_lo)
        return carry

    lax.fori_loop(0, _N_BINS // _LANES, fbody, 0)

    def do_chunk(k, b, g, res):
        chunk_copy(k, b).wait()
        zero = jnp.zeros((_LANES,), jnp.float32)

        def inner(j, accs):
            col = pl.multiple_of(j * _LANES, _LANES)
            f = flux_v[pl.ds(col, _LANES)]
            return tuple(acc + ring[b, i, pl.ds(col, _LANES)] * f
                         for i, acc in enumerate(accs))

        accs = lax.fori_loop(0, _N_BINS // _LANES, inner,
                             (zero,) * _RCHUNK)
        for i in range(_RCHUNK):
            s = plsc.cumsum(accs[i])[_LANES - 1]
            res = jnp.where(iota16 == g * _RCHUNK + i, s, res)

        @pl.when(k + _NBUF < n_chunks)
        def _():
            chunk_copy(k + _NBUF, b).start()
        return res

    def outer(grp, carry):
        k0 = grp * group
        res = jnp.zeros((_LANES,), jnp.float32)
        for j in range(group):
            res = do_chunk(k0 + j, j % _NBUF, j, res)
        base = pl.multiple_of(k0 * _RCHUNK, _LANES)
        out_v[pl.ds(base, _LANES)] = jnp.maximum(res, 1e-6)
        return carry

    lax.fori_loop(0, n_chunks // group, outer, 0)
    pltpu.sync_copy(out_v, out_hbm.at[pl.ds(wid * _SC_ROWS_PER_W,
                                            _SC_ROWS_PER_W)])


def _sc_matvec(parameters, energies, transfer_matrix):
    params16 = jnp.zeros((_LANES,), jnp.float32).at[:2].set(parameters)
    mesh = plsc.VectorSubcoreMesh(core_axis_name="c", subcore_axis_name="s")
    kern = functools.partial(
        pl.kernel,
        out_type=jax.ShapeDtypeStruct((_SC_ROWS,), jnp.float32),
        mesh=mesh,
        compiler_params=pltpu.CompilerParams(needs_layout_passes=False),
        scratch_types=[
            pltpu.VMEM((_LANES,), jnp.float32),
            pltpu.VMEM((2, _N_BINS), jnp.float32),
            pltpu.VMEM((_N_BINS,), jnp.float32),
            pltpu.VMEM((_NBUF, _RCHUNK, _N_BINS), jnp.float32),
            pltpu.VMEM((_SC_ROWS_PER_W,), jnp.float32),
            pltpu.SemaphoreType.DMA,
            pltpu.SemaphoreType.DMA,
        ],
    )(_sc_mv_body)
    return kern(params16, energies, transfer_matrix)


def kernel(parameters, energies, transfer_matrix):
    parts = []
    if _SC_ROWS:
        sc_part = _sc_matvec(parameters, energies, transfer_matrix)
    if _TC_ROWS:
        parts.append(_tc_matvec(parameters, energies, transfer_matrix))
    if _SC_ROWS:
        parts.append(sc_part)
    if len(parts) == 1:
        return parts[0]
    return jnp.concatenate(parts)


# speedup vs baseline: 1.0600x; 1.0025x over previous
import jax
import jax.numpy as jnp
from jax.experimental import pallas as pl
from jax.experimental.pallas import tpu as pltpu

_N_CHANNELS = 4096
_N_BINS = 8192
_ROW_BLOCK = 256
_N_ROW_BLOCKS = _N_CHANNELS // _ROW_BLOCK


def _mv_body(params_ref, energies_ref, m_ref, out_ref, flux_ref):
    i = pl.program_id(0)

    @pl.when(i == 0)
    def _():
        alpha = params_ref[0]
        norm = params_ref[1]
        oma = 1.0 - alpha
        e_low = energies_ref[0, :]
        e_high = energies_ref[1, :]
        flux_ref[0, :] = norm * (jnp.exp(oma * jnp.log(e_high))
                                 - jnp.exp(oma * jnp.log(e_low))) / oma

    m = m_ref[...]
    acc = jnp.sum(m * flux_ref[0, :][None, :], axis=1)
    out_ref[0, 0, :] = jnp.maximum(acc, 1e-6)


def kernel(parameters, energies, transfer_matrix):
    out = pl.pallas_call(
        _mv_body,
        grid=(_N_ROW_BLOCKS,),
        in_specs=[
            pl.BlockSpec(memory_space=pltpu.SMEM),
            pl.BlockSpec((2, _N_BINS), lambda i: (0, 0)),
            pl.BlockSpec((_ROW_BLOCK, _N_BINS), lambda i: (i, 0)),
        ],
        out_specs=pl.BlockSpec((1, 1, _ROW_BLOCK), lambda i: (i, 0, 0)),
        out_shape=jax.ShapeDtypeStruct((_N_ROW_BLOCKS, 1, _ROW_BLOCK),
                                       jnp.float32),
        scratch_shapes=[pltpu.VMEM((1, _N_BINS), jnp.float32)],
    )(parameters, energies, transfer_matrix)
    return out.reshape(_N_CHANNELS)
